# baseline (speedup 1.0000x reference)
import jax
import jax.numpy as jnp
from jax import lax
from jax.experimental import pallas as pl
from jax.experimental.pallas import tpu as pltpu
from jax.experimental.pallas import tpu_sc as plsc

HIDDEN = 64
ZDIM = 32
BATCH = 16384
WIDE_ROWS = 500000

_NC = 2
_NS = 16
_NW = _NC * _NS
_BPW = BATCH // _NW


def _gather_body(table_hbm, idx_hbm, out_hbm, idx_v, pair_v, rows_v, sem):
    wid = lax.axis_index("s") * _NC + lax.axis_index("c")
    base = wid * _BPW
    pltpu.sync_copy(idx_hbm.at[pl.ds(base, _BPW)], idx_v)

    def halve(g):
        v = idx_v[pl.ds(g * 16, 16)]
        pair_v[pl.ds(g * 16, 16)] = jax.lax.shift_right_logical(v, 1)

    pl.loop(0, _BPW // 16)(halve)

    pltpu.async_copy(table_hbm.at[pair_v], rows_v, sem).wait()
    pltpu.sync_copy(rows_v, out_hbm.at[pl.ds(base, _BPW)])


def _sc_gather_wide(table2, idx):
    mesh = plsc.VectorSubcoreMesh(core_axis_name="c", subcore_axis_name="s")
    f = pl.kernel(
        _gather_body,
        mesh=mesh,
        out_type=jax.ShapeDtypeStruct((BATCH, 2 * HIDDEN), jnp.float32),
        scratch_types=[
            pltpu.VMEM((_BPW,), jnp.int32),
            pltpu.VMEM((_BPW,), jnp.int32),
            pltpu.VMEM((_BPW, 2 * HIDDEN), jnp.float32),
            pltpu.SemaphoreType.DMA,
        ],
    )
    return f(table2, idx)


_BB = 2048


def _proj_body(e2_ref, c_ref, wmu_ref, bmu_ref, wlv_ref, blv_ref,
               mu_ref, lv_ref):
    e2 = e2_ref[...]
    odd = (c_ref[0, 0, :] & 1).reshape(_BB, 1) == 1
    e = jnp.where(odd, e2[:, HIDDEN:], e2[:, :HIDDEN])
    mu_ref[...] = (
        jnp.dot(e, wmu_ref[...], preferred_element_type=jnp.float32)
        + bmu_ref[...]
    )
    lv_ref[...] = (
        jnp.dot(e, wlv_ref[...], preferred_element_type=jnp.float32)
        + blv_ref[...]
    )


def _tc_proj(e2, c, W_mu, b_mu, W_lv, b_lv):
    grid = (BATCH // _BB,)
    c3 = c.reshape(BATCH // _BB, 1, _BB)
    return pl.pallas_call(
        _proj_body,
        grid=grid,
        in_specs=[
            pl.BlockSpec((_BB, 2 * HIDDEN), lambda i: (i, 0)),
            pl.BlockSpec((1, 1, _BB), lambda i: (i, 0, 0)),
            pl.BlockSpec((HIDDEN, ZDIM), lambda i: (0, 0)),
            pl.BlockSpec((1, ZDIM), lambda i: (0, 0)),
            pl.BlockSpec((HIDDEN, ZDIM), lambda i: (0, 0)),
            pl.BlockSpec((1, ZDIM), lambda i: (0, 0)),
        ],
        out_specs=[
            pl.BlockSpec((_BB, ZDIM), lambda i: (i, 0)),
            pl.BlockSpec((_BB, ZDIM), lambda i: (i, 0)),
        ],
        out_shape=[
            jax.ShapeDtypeStruct((BATCH, ZDIM), jnp.float32),
            jax.ShapeDtypeStruct((BATCH, ZDIM), jnp.float32),
        ],
    )(e2, c3, W_mu, b_mu.reshape(1, ZDIM), W_lv, b_lv.reshape(1, ZDIM))


def kernel(c, embedding, W_mu, b_mu, W_lv, b_lv):
    ci = c.astype(jnp.int32)
    table2 = embedding.reshape(WIDE_ROWS, 2 * HIDDEN)
    e2 = _sc_gather_wide(table2, ci)
    mu, lv = _tc_proj(e2, ci, W_mu, b_mu, W_lv, b_lv)
    return (mu, lv)


# speedup vs baseline: 1.3736x; 1.3736x over previous
import functools

import jax
import jax.numpy as jnp
from jax import lax
from jax.experimental import pallas as pl
from jax.experimental.pallas import tpu as pltpu
from jax.experimental.pallas import tpu_sc as plsc

HIDDEN = 64
ZDIM = 32
BATCH = 16384

_SC_ROWS = 4096
_NC = 2
_NS = 16
_NW = _NC * _NS
_BPW = _SC_ROWS // _NW

_TC_ROWS = BATCH - _SC_ROWS
_STEP = 512


def _sc_gather_body(table_hbm, idx_hbm, out_hbm, idx_v, rows_v, sem):
    wid = lax.axis_index("s") * _NC + lax.axis_index("c")
    base = wid * _BPW
    pltpu.sync_copy(idx_hbm.at[pl.ds(base, _BPW)], idx_v)

    def fire(g):
        v = idx_v[pl.ds(g * 16, 16)]
        for l in range(16):
            pltpu.async_copy(
                table_hbm.at[pl.ds(v[l], 1)],
                rows_v.at[pl.ds(g * 16 + l, 1)],
                sem,
            )

    pl.loop(0, _BPW // 16)(fire)

    def drain(g):
        pltpu.make_async_copy(
            table_hbm.at[pl.ds(0, 16)],
            rows_v.at[pl.ds(g * 16, 16)],
            sem,
        ).wait()

    pl.loop(0, _BPW // 16)(drain)
    pltpu.sync_copy(rows_v, out_hbm.at[pl.ds(base, _BPW)])


def _sc_gather(table, idx):
    mesh = plsc.VectorSubcoreMesh(core_axis_name="c", subcore_axis_name="s")
    f = pl.kernel(
        _sc_gather_body,
        mesh=mesh,
        out_type=jax.ShapeDtypeStruct((_SC_ROWS, HIDDEN), jnp.float32),
        scratch_types=[
            pltpu.VMEM((_BPW,), jnp.int32),
            pltpu.VMEM((_BPW, HIDDEN), jnp.float32),
            pltpu.SemaphoreType.DMA,
        ],
    )
    return f(table, idx)


def _tc_gather_proj_body(idx_ref, table_ref, wmu_ref, bmu_ref, wlv_ref,
                         blv_ref, mu_ref, lv_ref, buf, sem):
    g = pl.program_id(0)

    def fire(j, _):
        idx = idx_ref[_SC_ROWS + g * _STEP + j]
        pltpu.make_async_copy(
            table_ref.at[pl.ds(idx, 1)], buf.at[pl.ds(j, 1)], sem
        ).start()
        return 0

    lax.fori_loop(0, _STEP, fire, 0)
    pltpu.make_async_copy(table_ref.at[pl.ds(0, _STEP)], buf, sem).wait()
    e = buf[...]
    mu_ref[...] = (
        jnp.dot(e, wmu_ref[...], preferred_element_type=jnp.float32)
        + bmu_ref[...]
    )
    lv_ref[...] = (
        jnp.dot(e, wlv_ref[...], preferred_element_type=jnp.float32)
        + blv_ref[...]
    )


def _tc_gather_proj(c, table, W_mu, b_mu, W_lv, b_lv):
    grid_spec = pltpu.PrefetchScalarGridSpec(
        num_scalar_prefetch=1,
        grid=(_TC_ROWS // _STEP,),
        in_specs=[
            pl.BlockSpec(memory_space=pl.ANY),
            pl.BlockSpec((HIDDEN, ZDIM), lambda i, r: (0, 0)),
            pl.BlockSpec((1, ZDIM), lambda i, r: (0, 0)),
            pl.BlockSpec((HIDDEN, ZDIM), lambda i, r: (0, 0)),
            pl.BlockSpec((1, ZDIM), lambda i, r: (0, 0)),
        ],
        out_specs=[
            pl.BlockSpec((_STEP, ZDIM), lambda i, r: (i, 0)),
            pl.BlockSpec((_STEP, ZDIM), lambda i, r: (i, 0)),
        ],
        scratch_shapes=[
            pltpu.VMEM((_STEP, HIDDEN), jnp.float32),
            pltpu.SemaphoreType.DMA,
        ],
    )
    return pl.pallas_call(
        _tc_gather_proj_body,
        grid_spec=grid_spec,
        out_shape=[
            jax.ShapeDtypeStruct((_TC_ROWS, ZDIM), jnp.float32),
            jax.ShapeDtypeStruct((_TC_ROWS, ZDIM), jnp.float32),
        ],
    )(c, table, W_mu, b_mu.reshape(1, ZDIM), W_lv, b_lv.reshape(1, ZDIM))


_BB = 2048


def _proj_body(e_ref, wmu_ref, bmu_ref, wlv_ref, blv_ref, mu_ref, lv_ref):
    e = e_ref[...]
    mu_ref[...] = (
        jnp.dot(e, wmu_ref[...], preferred_element_type=jnp.float32)
        + bmu_ref[...]
    )
    lv_ref[...] = (
        jnp.dot(e, wlv_ref[...], preferred_element_type=jnp.float32)
        + blv_ref[...]
    )


def _tc_proj(e, W_mu, b_mu, W_lv, b_lv):
    n = e.shape[0]
    grid = (n // _BB,)
    return pl.pallas_call(
        _proj_body,
        grid=grid,
        in_specs=[
            pl.BlockSpec((_BB, HIDDEN), lambda i: (i, 0)),
            pl.BlockSpec((HIDDEN, ZDIM), lambda i: (0, 0)),
            pl.BlockSpec((1, ZDIM), lambda i: (0, 0)),
            pl.BlockSpec((HIDDEN, ZDIM), lambda i: (0, 0)),
            pl.BlockSpec((1, ZDIM), lambda i: (0, 0)),
        ],
        out_specs=[
            pl.BlockSpec((_BB, ZDIM), lambda i: (i, 0)),
            pl.BlockSpec((_BB, ZDIM), lambda i: (i, 0)),
        ],
        out_shape=[
            jax.ShapeDtypeStruct((n, ZDIM), jnp.float32),
            jax.ShapeDtypeStruct((n, ZDIM), jnp.float32),
        ],
    )(e, W_mu, b_mu.reshape(1, ZDIM), W_lv, b_lv.reshape(1, ZDIM))


def kernel(c, embedding, W_mu, b_mu, W_lv, b_lv):
    ci = c.astype(jnp.int32)
    e_sc = _sc_gather(embedding, ci)
    mu_tc, lv_tc = _tc_gather_proj(ci, embedding, W_mu, b_mu, W_lv, b_lv)
    mu_sc, lv_sc = _tc_proj(e_sc, W_mu, b_mu, W_lv, b_lv)
    mu = jnp.concatenate([mu_sc, mu_tc], axis=0)
    lv = jnp.concatenate([lv_sc, lv_tc], axis=0)
    return (mu, lv)


# speedup vs baseline: 1.6785x; 1.2220x over previous
import functools

import jax
import jax.numpy as jnp
from jax import lax
from jax.experimental import pallas as pl
from jax.experimental.pallas import tpu as pltpu
from jax.experimental.pallas import tpu_sc as plsc

HIDDEN = 64
ZDIM = 32
BATCH = 16384

_NC = 2
_NS = 16
_NW = _NC * _NS
_BPW = BATCH // _NW


def _gather_body(table_hbm, idx_hbm, out_hbm, idx_v, rows_v, sem):
    wid = lax.axis_index("s") * _NC + lax.axis_index("c")
    base = wid * _BPW
    pltpu.sync_copy(idx_hbm.at[pl.ds(base, _BPW)], idx_v)

    def fire(g):
        v = idx_v[pl.ds(g * 16, 16)]
        for l in range(16):
            pltpu.async_copy(
                table_hbm.at[pl.ds(v[l], 1)],
                rows_v.at[pl.ds(g * 16 + l, 1)],
                sem,
            )

    pl.loop(0, _BPW // 16)(fire)

    def drain(j):
        pltpu.make_async_copy(
            table_hbm.at[pl.ds(0, 1)], rows_v.at[pl.ds(j, 1)], sem
        ).wait()

    pl.loop(0, _BPW)(drain)
    pltpu.sync_copy(rows_v, out_hbm.at[pl.ds(base, _BPW)])


def _sc_gather(table, idx):
    mesh = plsc.VectorSubcoreMesh(core_axis_name="c", subcore_axis_name="s")
    f = pl.kernel(
        _gather_body,
        mesh=mesh,
        out_type=jax.ShapeDtypeStruct((BATCH, HIDDEN), jnp.float32),
        scratch_types=[
            pltpu.VMEM((_BPW,), jnp.int32),
            pltpu.VMEM((_BPW, HIDDEN), jnp.float32),
            pltpu.SemaphoreType.DMA,
        ],
    )
    return f(table, idx)


_BB = 2048


def _proj_body(e_ref, wmu_ref, bmu_ref, wlv_ref, blv_ref, mu_ref, lv_ref):
    e = e_ref[...]
    mu_ref[...] = (
        jnp.dot(e, wmu_ref[...], preferred_element_type=jnp.float32)
        + bmu_ref[...]
    )
    lv_ref[...] = (
        jnp.dot(e, wlv_ref[...], preferred_element_type=jnp.float32)
        + blv_ref[...]
    )


def _tc_proj(e, W_mu, b_mu, W_lv, b_lv):
    grid = (BATCH // _BB,)
    return pl.pallas_call(
        _proj_body,
        grid=grid,
        in_specs=[
            pl.BlockSpec((_BB, HIDDEN), lambda i: (i, 0)),
            pl.BlockSpec((HIDDEN, ZDIM), lambda i: (0, 0)),
            pl.BlockSpec((1, ZDIM), lambda i: (0, 0)),
            pl.BlockSpec((HIDDEN, ZDIM), lambda i: (0, 0)),
            pl.BlockSpec((1, ZDIM), lambda i: (0, 0)),
        ],
        out_specs=[
            pl.BlockSpec((_BB, ZDIM), lambda i: (i, 0)),
            pl.BlockSpec((_BB, ZDIM), lambda i: (i, 0)),
        ],
        out_shape=[
            jax.ShapeDtypeStruct((BATCH, ZDIM), jnp.float32),
            jax.ShapeDtypeStruct((BATCH, ZDIM), jnp.float32),
        ],
    )(e, W_mu, b_mu.reshape(1, ZDIM), W_lv, b_lv.reshape(1, ZDIM))


def kernel(c, embedding, W_mu, b_mu, W_lv, b_lv):
    e = _sc_gather(embedding, c.astype(jnp.int32))
    mu, lv = _tc_proj(e, W_mu, b_mu, W_lv, b_lv)
    return (mu, lv)
